# baseline (speedup 1.0000x reference)
import functools

import jax
import jax.numpy as jnp
from jax import lax
from jax.experimental import pallas as pl
from jax.experimental.pallas import tpu as pltpu
from jax.experimental.pallas import tpu_sc as plsc

_NC = 2
_NS = 16
_K = 128
_G = 128


@functools.partial(jax.jit, static_argnames=("np_", "hd", "nch"))
def _seg_sum_sc(h, idx4, zeros, *, np_, hd, nch):
  rows_per_tile = np_ // _NS

  nb = 2 if hd > 64 else 4

  def body(h_hbm, idx_hbm, zeros_hbm, out_hbm, *refs):
    idx_vs = refs[0:nb]
    rows_vs = refs[nb:2 * nb]
    agg_sh = refs[2 * nb]
    gsems = refs[2 * nb + 1:3 * nb + 1]
    ssems = refs[3 * nb + 1:4 * nb + 1]
    cid = lax.axis_index("c")
    sid = lax.axis_index("s")
    wid = cid * _NS + sid
    row0 = sid * rows_per_tile
    pltpu.sync_copy(zeros_hbm.at[pl.ds(row0, rows_per_tile)],
                    agg_sh.at[pl.ds(row0, rows_per_tile)])

    def fire(b, c):
      pltpu.sync_copy(idx_hbm.at[wid, c], idx_vs[b])
      pltpu.async_copy(h_hbm.at[idx_vs[b].at[0]], rows_vs[b], gsems[b])

    def proc(b):
      pltpu.make_async_copy(h_hbm.at[idx_vs[b].at[0]], rows_vs[b],
                            gsems[b]).wait()
      pltpu.async_copy(rows_vs[b], agg_sh.at[idx_vs[b].at[1]], ssems[b],
                       add=True)

    def reuse(b):
      pltpu.make_async_copy(rows_vs[b], agg_sh.at[idx_vs[b].at[1]],
                            ssems[b]).wait()

    plsc.subcore_barrier()
    for b in range(nb):
      fire(b, b)

    def step(i, _):
      c = nb * i
      for b in range(nb):
        proc(b)
      for b in range(nb):
        reuse(b)
        fire(b, c + nb + b)
      return 0

    lax.fori_loop(0, nch // nb - 1, step, 0)
    for b in range(nb):
      proc(b)
    for b in range(nb):
      reuse(b)
    plsc.subcore_barrier()
    pltpu.sync_copy(agg_sh.at[pl.ds(row0, rows_per_tile)],
                    out_hbm.at[cid, pl.ds(row0, rows_per_tile)])

  fn = pl.kernel(
      body,
      out_type=jax.ShapeDtypeStruct((_NC, np_, hd), jnp.float32),
      mesh=plsc.VectorSubcoreMesh(core_axis_name="c", subcore_axis_name="s",
                                  num_cores=_NC, num_subcores=_NS),
      scratch_types=(
          [pltpu.VMEM((2, _K), jnp.int32) for _ in range(nb)]
          + [pltpu.VMEM((_K, hd), jnp.float32) for _ in range(nb)]
          + [pltpu.VMEM_SHARED((np_, hd), jnp.float32)]
          + [pltpu.SemaphoreType.DMA for _ in range(2 * nb)]
      ),
      compiler_params=pltpu.CompilerParams(use_tc_tiling_on_sc=False),
  )
  return fn(h, idx4, zeros)


def _layer_body(h_ref, s_ref, w1_ref, b1_ref, w2_ref, b2_ref, g_ref, bb_ref,
                o_ref):
  n = h_ref.shape[0]
  z = h_ref[...] + s_ref[0, :n] + s_ref[1, :n]
  z1 = jnp.maximum(jnp.dot(z, w1_ref[...],
                           preferred_element_type=jnp.float32) + b1_ref[...],
                   0.0)
  z2 = jnp.dot(z1, w2_ref[...],
               preferred_element_type=jnp.float32) + b2_ref[...]
  r = jnp.maximum(z2, 0.0)
  mean = jnp.mean(r, axis=0, keepdims=True)
  var = jnp.mean((r - mean) ** 2, axis=0, keepdims=True)
  o_ref[...] = (r - mean) / jnp.sqrt(var + 1e-5) * g_ref[...] + bb_ref[...]


def _layer(h, s, w1, b1, w2, b2, g, bb):
  return pl.pallas_call(
      _layer_body,
      out_shape=jax.ShapeDtypeStruct((h.shape[0], w2.shape[1]), jnp.float32),
  )(h, s, w1, b1, w2, b2, g, bb)


def _final_body(h_ref, batch_ref, f1w_ref, f1b_ref, f2w_ref, f2b_ref, o_ref):
  n = h_ref.shape[0]
  onehot = (batch_ref[...] ==
            lax.broadcasted_iota(jnp.int32, (n, _G), 1)).astype(jnp.float32)
  pooled = lax.dot_general(onehot, h_ref[...], (((0,), (0,)), ((), ())),
                           precision=lax.Precision.HIGHEST,
                           preferred_element_type=jnp.float32)
  z = jnp.maximum(jnp.dot(pooled, f1w_ref[...],
                          preferred_element_type=jnp.float32) + f1b_ref[...],
                  0.0)
  z = jnp.dot(z, f2w_ref[...],
              preferred_element_type=jnp.float32) + f2b_ref[...]
  m = jnp.max(z, axis=-1, keepdims=True)
  lse = jnp.log(jnp.sum(jnp.exp(z - m), axis=-1, keepdims=True)) + m
  o_ref[...] = z - lse


def _final(h, batch2, f1w, f1b, f2w, f2b):
  return pl.pallas_call(
      _final_body,
      out_shape=jax.ShapeDtypeStruct((_G, f2w.shape[1]), jnp.float32),
  )(h, batch2, f1w, f1b, f2w, f2b)


def kernel(x, edge_index, batch, params):
  n, d = x.shape
  e = edge_index.shape[1]
  h = params["conv1_w1"].shape[1]
  nw = _NC * _NS
  np_ = (n + 1 + _NS * 8 - 1) // (_NS * 8) * (_NS * 8)
  nch = -(-e // (nw * _K))
  nch = -(-nch // 4) * 4
  ep = nw * nch * _K
  pad = ep - e
  src_p = jnp.concatenate([edge_index[0], jnp.zeros((pad,), jnp.int32)])
  dst_p = jnp.concatenate([edge_index[1],
                           jnp.full((pad,), np_ - 1, jnp.int32)])
  idx4 = jnp.stack([src_p.reshape(nw, nch, _K),
                    dst_p.reshape(nw, nch, _K)], axis=2)
  zeros_d = jnp.zeros((np_, d), jnp.float32)
  zeros_h = jnp.zeros((np_, h), jnp.float32)
  batch2 = batch.reshape(n, 1)
  row = lambda v: v.reshape(1, -1)

  hcur = x
  for l in range(1, 6):
    zeros = zeros_d if hcur.shape[1] == d else zeros_h
    s = _seg_sum_sc(hcur, idx4, zeros, np_=np_, hd=hcur.shape[1], nch=nch)
    hcur = _layer(hcur, s, params[f"conv{l}_w1"], row(params[f"conv{l}_b1"]),
                  params[f"conv{l}_w2"], row(params[f"conv{l}_b2"]),
                  row(params[f"bn{l}_g"]), row(params[f"bn{l}_b"]))
  return _final(hcur, batch2, params["fc1_w"], row(params["fc1_b"]),
                params["fc2_w"], row(params["fc2_b"]))


# speedup vs baseline: 1.0681x; 1.0681x over previous
import functools

import jax
import jax.numpy as jnp
from jax import lax
from jax.experimental import pallas as pl
from jax.experimental.pallas import tpu as pltpu
from jax.experimental.pallas import tpu_sc as plsc

_NC = 2
_NS = 16
_K = 128
_G = 128


@functools.partial(jax.jit, static_argnames=("np_", "hd", "nch"))
def _seg_sum_sc(h, idx4, zeros, *, np_, hd, nch):
  rows_per_tile = np_ // _NS

  preload = hd <= 64

  def body(h_hbm, idx_hbm, zeros_hbm, out_hbm, *refs):
    if preload:
      idx_all, = refs[0:1]
      idx_vs = None
      rest = refs[1:]
    else:
      idx_vs = refs[0:2]
      rest = refs[2:]
    rows_vs = rest[0:2]
    agg_sh = rest[2]
    gsems = rest[3:5]
    cid = lax.axis_index("c")
    sid = lax.axis_index("s")
    wid = cid * _NS + sid
    row0 = sid * rows_per_tile
    pltpu.sync_copy(zeros_hbm.at[pl.ds(row0, rows_per_tile)],
                    agg_sh.at[pl.ds(row0, rows_per_tile)])
    if preload:
      pltpu.sync_copy(idx_hbm.at[wid], idx_all)

    def idxr(b, c):
      return idx_all.at[c] if preload else idx_vs[b]

    def fire(b, c):
      if not preload:
        pltpu.sync_copy(idx_hbm.at[wid, c], idx_vs[b])
      pltpu.async_copy(h_hbm.at[idxr(b, c).at[0]], rows_vs[b], gsems[b])

    def drain(b, c):
      pltpu.make_async_copy(h_hbm.at[idxr(b, c).at[0]], rows_vs[b],
                            gsems[b]).wait()
      pltpu.sync_copy(rows_vs[b], agg_sh.at[idxr(b, c).at[1]], add=True)

    plsc.subcore_barrier()
    fire(0, 0)
    fire(1, 1)

    def step(i, _):
      c = 2 * i
      drain(0, c)
      fire(0, c + 2)
      drain(1, c + 1)
      fire(1, c + 3)
      return 0

    lax.fori_loop(0, nch // 2 - 1, step, 0)
    drain(0, nch - 2)
    drain(1, nch - 1)
    plsc.subcore_barrier()
    pltpu.sync_copy(agg_sh.at[pl.ds(row0, rows_per_tile)],
                    out_hbm.at[cid, pl.ds(row0, rows_per_tile)])

  fn = pl.kernel(
      body,
      out_type=jax.ShapeDtypeStruct((_NC, np_, hd), jnp.float32),
      mesh=plsc.VectorSubcoreMesh(core_axis_name="c", subcore_axis_name="s",
                                  num_cores=_NC, num_subcores=_NS),
      scratch_types=(
          ([pltpu.VMEM((nch, 2, _K), jnp.int32)] if preload else
           [pltpu.VMEM((2, _K), jnp.int32) for _ in range(2)])
          + [pltpu.VMEM((_K, hd), jnp.float32) for _ in range(2)]
          + [pltpu.VMEM_SHARED((np_, hd), jnp.float32)]
          + [pltpu.SemaphoreType.DMA for _ in range(2)]
      ),
      compiler_params=pltpu.CompilerParams(use_tc_tiling_on_sc=False),
  )
  return fn(h, idx4, zeros)


def _layer_body(h_ref, s_ref, w1_ref, b1_ref, w2_ref, b2_ref, g_ref, bb_ref,
                o_ref):
  n = h_ref.shape[0]
  z = h_ref[...] + s_ref[0, :n] + s_ref[1, :n]
  z1 = jnp.maximum(jnp.dot(z, w1_ref[...],
                           preferred_element_type=jnp.float32) + b1_ref[...],
                   0.0)
  z2 = jnp.dot(z1, w2_ref[...],
               preferred_element_type=jnp.float32) + b2_ref[...]
  r = jnp.maximum(z2, 0.0)
  mean = jnp.mean(r, axis=0, keepdims=True)
  var = jnp.mean((r - mean) ** 2, axis=0, keepdims=True)
  o_ref[...] = (r - mean) / jnp.sqrt(var + 1e-5) * g_ref[...] + bb_ref[...]


def _layer(h, s, w1, b1, w2, b2, g, bb):
  return pl.pallas_call(
      _layer_body,
      out_shape=jax.ShapeDtypeStruct((h.shape[0], w2.shape[1]), jnp.float32),
  )(h, s, w1, b1, w2, b2, g, bb)


def _final_body(h_ref, batch_ref, f1w_ref, f1b_ref, f2w_ref, f2b_ref, o_ref):
  n = h_ref.shape[0]
  onehot = (batch_ref[...] ==
            lax.broadcasted_iota(jnp.int32, (n, _G), 1)).astype(jnp.float32)
  pooled = lax.dot_general(onehot, h_ref[...], (((0,), (0,)), ((), ())),
                           precision=lax.Precision.HIGHEST,
                           preferred_element_type=jnp.float32)
  z = jnp.maximum(jnp.dot(pooled, f1w_ref[...],
                          preferred_element_type=jnp.float32) + f1b_ref[...],
                  0.0)
  z = jnp.dot(z, f2w_ref[...],
              preferred_element_type=jnp.float32) + f2b_ref[...]
  m = jnp.max(z, axis=-1, keepdims=True)
  lse = jnp.log(jnp.sum(jnp.exp(z - m), axis=-1, keepdims=True)) + m
  o_ref[...] = z - lse


def _final(h, batch2, f1w, f1b, f2w, f2b):
  return pl.pallas_call(
      _final_body,
      out_shape=jax.ShapeDtypeStruct((_G, f2w.shape[1]), jnp.float32),
  )(h, batch2, f1w, f1b, f2w, f2b)


def kernel(x, edge_index, batch, params):
  n, d = x.shape
  e = edge_index.shape[1]
  h = params["conv1_w1"].shape[1]
  nw = _NC * _NS
  np_ = (n + 1 + _NS * 8 - 1) // (_NS * 8) * (_NS * 8)
  nch = -(-e // (nw * _K))
  nch = -(-nch // 4) * 4
  ep = nw * nch * _K
  pad = ep - e
  src_p = jnp.concatenate([edge_index[0], jnp.zeros((pad,), jnp.int32)])
  dst_p = jnp.concatenate([edge_index[1],
                           jnp.full((pad,), np_ - 1, jnp.int32)])
  idx4 = jnp.stack([src_p.reshape(nw, nch, _K),
                    dst_p.reshape(nw, nch, _K)], axis=2)
  zeros_d = jnp.zeros((np_, d), jnp.float32)
  zeros_h = jnp.zeros((np_, h), jnp.float32)
  batch2 = batch.reshape(n, 1)
  row = lambda v: v.reshape(1, -1)

  hcur = x
  for l in range(1, 6):
    zeros = zeros_d if hcur.shape[1] == d else zeros_h
    s = _seg_sum_sc(hcur, idx4, zeros, np_=np_, hd=hcur.shape[1], nch=nch)
    hcur = _layer(hcur, s, params[f"conv{l}_w1"], row(params[f"conv{l}_b1"]),
                  params[f"conv{l}_w2"], row(params[f"conv{l}_b2"]),
                  row(params[f"bn{l}_g"]), row(params[f"bn{l}_b"]))
  return _final(hcur, batch2, params["fc1_w"], row(params["fc1_b"]),
                params["fc2_w"], row(params["fc2_b"]))


# speedup vs baseline: 1.0790x; 1.0103x over previous
import functools

import jax
import jax.numpy as jnp
from jax import lax
from jax.experimental import pallas as pl
from jax.experimental.pallas import tpu as pltpu
from jax.experimental.pallas import tpu_sc as plsc

_NC = 2
_NS = 16
_K = 128
_G = 128


@functools.partial(jax.jit, static_argnames=("np_", "hd", "nch"))
def _seg_sum_sc(h, idx4, zeros, *, np_, hd, nch):
  rows_per_tile = np_ // _NS

  preload = hd <= 64
  nbuf = 4 if preload else 2

  def body(h_hbm, idx_hbm, zeros_hbm, out_hbm, *refs):
    if preload:
      idx_all, = refs[0:1]
      idx_vs = None
      rest = refs[1:]
    else:
      idx_vs = refs[0:2]
      rest = refs[2:]
    rows_vs = rest[0:nbuf]
    agg_sh = rest[nbuf]
    gsems = rest[nbuf + 1:2 * nbuf + 1]
    cid = lax.axis_index("c")
    sid = lax.axis_index("s")
    wid = cid * _NS + sid
    row0 = sid * rows_per_tile
    pltpu.sync_copy(zeros_hbm.at[pl.ds(row0, rows_per_tile)],
                    agg_sh.at[pl.ds(row0, rows_per_tile)])
    if preload:
      pltpu.sync_copy(idx_hbm.at[wid], idx_all)

    def idxr(b, c):
      return idx_all.at[c] if preload else idx_vs[b]

    def fire(b, c):
      if not preload:
        pltpu.sync_copy(idx_hbm.at[wid, c], idx_vs[b])
      pltpu.async_copy(h_hbm.at[idxr(b, c).at[0]], rows_vs[b], gsems[b])

    def drain(b, c):
      pltpu.make_async_copy(h_hbm.at[idxr(b, c).at[0]], rows_vs[b],
                            gsems[b]).wait()
      pltpu.sync_copy(rows_vs[b], agg_sh.at[idxr(b, c).at[1]], add=True)

    plsc.subcore_barrier()
    for b in range(nbuf):
      fire(b, b)

    def step(i, _):
      c = nbuf * i
      for b in range(nbuf):
        drain(b, c + b)
        fire(b, c + nbuf + b)
      return 0

    lax.fori_loop(0, nch // nbuf - 1, step, 0)
    for b in range(nbuf):
      drain(b, nch - nbuf + b)
    plsc.subcore_barrier()
    pltpu.sync_copy(agg_sh.at[pl.ds(row0, rows_per_tile)],
                    out_hbm.at[cid, pl.ds(row0, rows_per_tile)])

  fn = pl.kernel(
      body,
      out_type=jax.ShapeDtypeStruct((_NC, np_, hd), jnp.float32),
      mesh=plsc.VectorSubcoreMesh(core_axis_name="c", subcore_axis_name="s",
                                  num_cores=_NC, num_subcores=_NS),
      scratch_types=(
          ([pltpu.VMEM((nch, 2, _K), jnp.int32)] if preload else
           [pltpu.VMEM((2, _K), jnp.int32) for _ in range(nbuf)])
          + [pltpu.VMEM((_K, hd), jnp.float32) for _ in range(nbuf)]
          + [pltpu.VMEM_SHARED((np_, hd), jnp.float32)]
          + [pltpu.SemaphoreType.DMA for _ in range(nbuf)]
      ),
      compiler_params=pltpu.CompilerParams(use_tc_tiling_on_sc=False),
  )
  return fn(h, idx4, zeros)


def _layer_body(h_ref, s_ref, w1_ref, b1_ref, w2_ref, b2_ref, g_ref, bb_ref,
                o_ref):
  n = h_ref.shape[0]
  z = h_ref[...] + s_ref[0, :n] + s_ref[1, :n]
  z1 = jnp.maximum(jnp.dot(z, w1_ref[...],
                           preferred_element_type=jnp.float32) + b1_ref[...],
                   0.0)
  z2 = jnp.dot(z1, w2_ref[...],
               preferred_element_type=jnp.float32) + b2_ref[...]
  r = jnp.maximum(z2, 0.0)
  mean = jnp.mean(r, axis=0, keepdims=True)
  var = jnp.mean((r - mean) ** 2, axis=0, keepdims=True)
  o_ref[...] = (r - mean) / jnp.sqrt(var + 1e-5) * g_ref[...] + bb_ref[...]


def _layer(h, s, w1, b1, w2, b2, g, bb):
  return pl.pallas_call(
      _layer_body,
      out_shape=jax.ShapeDtypeStruct((h.shape[0], w2.shape[1]), jnp.float32),
  )(h, s, w1, b1, w2, b2, g, bb)


def _final_body(h_ref, batch_ref, f1w_ref, f1b_ref, f2w_ref, f2b_ref, o_ref):
  n = h_ref.shape[0]
  onehot = (batch_ref[...] ==
            lax.broadcasted_iota(jnp.int32, (n, _G), 1)).astype(jnp.float32)
  pooled = lax.dot_general(onehot, h_ref[...], (((0,), (0,)), ((), ())),
                           precision=lax.Precision.HIGHEST,
                           preferred_element_type=jnp.float32)
  z = jnp.maximum(jnp.dot(pooled, f1w_ref[...],
                          preferred_element_type=jnp.float32) + f1b_ref[...],
                  0.0)
  z = jnp.dot(z, f2w_ref[...],
              preferred_element_type=jnp.float32) + f2b_ref[...]
  m = jnp.max(z, axis=-1, keepdims=True)
  lse = jnp.log(jnp.sum(jnp.exp(z - m), axis=-1, keepdims=True)) + m
  o_ref[...] = z - lse


def _final(h, batch2, f1w, f1b, f2w, f2b):
  return pl.pallas_call(
      _final_body,
      out_shape=jax.ShapeDtypeStruct((_G, f2w.shape[1]), jnp.float32),
  )(h, batch2, f1w, f1b, f2w, f2b)


def kernel(x, edge_index, batch, params):
  n, d = x.shape
  e = edge_index.shape[1]
  h = params["conv1_w1"].shape[1]
  nw = _NC * _NS
  np_ = (n + 1 + _NS * 8 - 1) // (_NS * 8) * (_NS * 8)
  nch = -(-e // (nw * _K))
  nch = -(-nch // 4) * 4
  ep = nw * nch * _K
  pad = ep - e
  src_p = jnp.concatenate([edge_index[0], jnp.zeros((pad,), jnp.int32)])
  dst_p = jnp.concatenate([edge_index[1],
                           jnp.full((pad,), np_ - 1, jnp.int32)])
  idx4 = jnp.stack([src_p.reshape(nw, nch, _K),
                    dst_p.reshape(nw, nch, _K)], axis=2)
  zeros_d = jnp.zeros((np_, d), jnp.float32)
  zeros_h = jnp.zeros((np_, h), jnp.float32)
  batch2 = batch.reshape(n, 1)
  row = lambda v: v.reshape(1, -1)

  hcur = x
  for l in range(1, 6):
    zeros = zeros_d if hcur.shape[1] == d else zeros_h
    s = _seg_sum_sc(hcur, idx4, zeros, np_=np_, hd=hcur.shape[1], nch=nch)
    hcur = _layer(hcur, s, params[f"conv{l}_w1"], row(params[f"conv{l}_b1"]),
                  params[f"conv{l}_w2"], row(params[f"conv{l}_b2"]),
                  row(params[f"bn{l}_g"]), row(params[f"bn{l}_b"]))
  return _final(hcur, batch2, params["fc1_w"], row(params["fc1_b"]),
                params["fc2_w"], row(params["fc2_b"]))
